# Initial kernel scaffold; baseline (speedup 1.0000x reference)
import jax
import jax.numpy as jnp
from jax.experimental import pallas as pl


def kernel(input, mask):
    raise NotImplementedError("write your pallas kernel here")


# speedup vs baseline: 1.7061x; 1.7061x over previous
import functools

import jax
import jax.numpy as jnp
from jax import lax
from jax.experimental import pallas as pl
from jax.experimental.pallas import tpu as pltpu
from jax.experimental.pallas import tpu_sc as plsc

B, N, D = 16, 4096, 256
ROWS = B * N
NW = 32
ROWS_PER_W = ROWS // NW
CHUNK = 128
NCHUNK = ROWS_PER_W // CHUNK


def _fill_index_body(mask_ref, gidx_ref):
    m = mask_ref[...]
    pos = lax.broadcasted_iota(jnp.int32, (B, N), 1)
    valid = jnp.where(m > 0.9, jnp.int32(-1), pos)
    ff = valid
    s = 1
    while s < N:
        shifted = jnp.concatenate(
            [jnp.full((B, s), -1, jnp.int32), ff[:, : N - s]], axis=1)
        ff = jnp.maximum(ff, shifted)
        s *= 2
    last = jnp.max(valid, axis=1, keepdims=True)
    fill = jnp.where(ff >= 0, ff, jnp.broadcast_to(last, (B, N)))
    fill = jnp.maximum(fill, 0)
    row = lax.broadcasted_iota(jnp.int32, (B, N), 0)
    gidx_ref[...] = fill + row * N


_fill_index = pl.pallas_call(
    _fill_index_body,
    out_shape=jax.ShapeDtypeStruct((B, N), jnp.int32),
)


@functools.cache
def _make_sc_gather():
    mesh = plsc.VectorSubcoreMesh(core_axis_name="c", subcore_axis_name="s")

    @functools.partial(
        pl.kernel,
        mesh=mesh,
        out_type=jax.ShapeDtypeStruct((ROWS, D), jnp.float32),
        scratch_types=[
            pltpu.VMEM((NCHUNK, CHUNK), jnp.int32),
            pltpu.VMEM((CHUNK, D), jnp.float32),
            pltpu.VMEM((CHUNK, D), jnp.float32),
            pltpu.SemaphoreType.DMA,
            pltpu.SemaphoreType.DMA,
        ],
    )
    def sc_gather(x_hbm, idx_hbm, out_hbm, idx_v, buf0, buf1, sem0, sem1):
        w = lax.axis_index("s") * 2 + lax.axis_index("c")
        pltpu.sync_copy(idx_hbm.at[w], idx_v)
        bufs = (buf0, buf1)
        sems = (sem0, sem1)
        copies = [None, None]
        copies[0] = pltpu.async_copy(x_hbm.at[idx_v.at[0]], buf0, sem0)
        for c in range(NCHUNK):
            if c + 1 < NCHUNK:
                nxt = (c + 1) % 2
                copies[nxt] = pltpu.async_copy(
                    x_hbm.at[idx_v.at[c + 1]], bufs[nxt], sems[nxt])
            copies[c % 2].wait()
            base = (w * NCHUNK + c) * CHUNK
            pltpu.sync_copy(bufs[c % 2], out_hbm.at[pl.ds(base, CHUNK)])

    return sc_gather


def kernel(input, mask):
    gidx = _fill_index(mask)
    gidx3 = gidx.reshape(NW, NCHUNK, CHUNK)
    x2d = input.reshape(ROWS, D)
    out = _make_sc_gather()(x2d, gidx3)
    return out.reshape(B, N, D)


# speedup vs baseline: 1.7240x; 1.0104x over previous
import functools

import jax
import jax.numpy as jnp
from jax import lax
from jax.experimental import pallas as pl
from jax.experimental.pallas import tpu as pltpu
from jax.experimental.pallas import tpu_sc as plsc

B, N, D = 16, 4096, 256
ROWS = B * N
NW = 32
ROWS_PER_W = ROWS // NW
CHUNK = 128
NCHUNK = ROWS_PER_W // CHUNK


def _fill_index_body(mask_ref, gidx_ref):
    m = mask_ref[...]
    pos = lax.broadcasted_iota(jnp.int32, (B, N), 1)
    valid = jnp.where(m > 0.9, jnp.int32(-1), pos)
    ff = valid
    s = 1
    while s < N:
        shifted = jnp.concatenate(
            [jnp.full((B, s), -1, jnp.int32), ff[:, : N - s]], axis=1)
        ff = jnp.maximum(ff, shifted)
        s *= 2
    last = jnp.max(valid, axis=1, keepdims=True)
    fill = jnp.where(ff >= 0, ff, jnp.broadcast_to(last, (B, N)))
    fill = jnp.maximum(fill, 0)
    row = lax.broadcasted_iota(jnp.int32, (B, N), 0)
    gidx_ref[...] = fill + row * N


_fill_index = pl.pallas_call(
    _fill_index_body,
    out_shape=jax.ShapeDtypeStruct((B, N), jnp.int32),
)


@functools.cache
def _make_sc_gather():
    mesh = plsc.VectorSubcoreMesh(core_axis_name="c", subcore_axis_name="s")

    @functools.partial(
        pl.kernel,
        mesh=mesh,
        out_type=jax.ShapeDtypeStruct((ROWS, D), jnp.float32),
        scratch_types=[
            pltpu.VMEM((NCHUNK, CHUNK), jnp.int32),
            pltpu.VMEM((CHUNK, D), jnp.float32),
            pltpu.VMEM((CHUNK, D), jnp.float32),
            pltpu.VMEM((CHUNK, D), jnp.float32),
            pltpu.SemaphoreType.DMA,
            pltpu.SemaphoreType.DMA,
            pltpu.SemaphoreType.DMA,
            pltpu.SemaphoreType.DMA,
            pltpu.SemaphoreType.DMA,
            pltpu.SemaphoreType.DMA,
        ],
    )
    def sc_gather(x_hbm, idx_hbm, out_hbm, idx_v,
                  b0, b1, b2, g0, g1, g2, w0, w1, w2):
        NB = 3
        w = lax.axis_index("s") * 2 + lax.axis_index("c")
        pltpu.sync_copy(idx_hbm.at[w], idx_v)
        bufs = (b0, b1, b2)
        gsems = (g0, g1, g2)
        wsems = (w0, w1, w2)
        gcp = [None] * NCHUNK
        wcp = [None] * NCHUNK
        for c in range(NB):
            gcp[c] = pltpu.async_copy(x_hbm.at[idx_v.at[c]], bufs[c], gsems[c])
        for c in range(NCHUNK):
            b = c % NB
            gcp[c].wait()
            base = (w * NCHUNK + c) * CHUNK
            wcp[c] = pltpu.async_copy(
                bufs[b], out_hbm.at[pl.ds(base, CHUNK)], wsems[b])
            nxt = c + NB
            if nxt < NCHUNK:
                wcp[c].wait()
                gcp[nxt] = pltpu.async_copy(
                    x_hbm.at[idx_v.at[nxt]], bufs[b], gsems[b])
        for c in range(NCHUNK - NB, NCHUNK):
            wcp[c].wait()

    return sc_gather


def kernel(input, mask):
    gidx = _fill_index(mask)
    gidx3 = gidx.reshape(NW, NCHUNK, CHUNK)
    x2d = input.reshape(ROWS, D)
    out = _make_sc_gather()(x2d, gidx3)
    return out.reshape(B, N, D)
